# Initial kernel scaffold; baseline (speedup 1.0000x reference)
import jax
import jax.numpy as jnp
from jax.experimental import pallas as pl


def kernel(x):
    raise NotImplementedError("write your pallas kernel here")


# speedup vs baseline: 3.5757x; 3.5757x over previous
import jax
import jax.numpy as jnp
from jax.experimental import pallas as pl


def _pool_body(x_ref, o_ref):
    x = x_ref[0]
    xc = jnp.clip(x, -20.0, 20.0)
    e = jnp.exp(xc - 20.0)
    s = jnp.sum(e, axis=0)
    w = jnp.sum(e * x, axis=0)
    o_ref[0, 0] = w / s


def kernel(x):
    B, C, H = x.shape
    HT = 512
    out = pl.pallas_call(
        _pool_body,
        grid=(B, H // HT),
        in_specs=[pl.BlockSpec((1, C, HT), lambda b, h: (b, 0, h))],
        out_specs=pl.BlockSpec((1, 1, HT), lambda b, h: (b, 0, h)),
        out_shape=jax.ShapeDtypeStruct((B, 1, H), x.dtype),
    )(x)
    return out.reshape(B, H)
